# Initial kernel scaffold; baseline (speedup 1.0000x reference)
import jax
import jax.numpy as jnp
from jax.experimental import pallas as pl


def kernel(x, V_raw, U_raw, u_scales):
    raise NotImplementedError("write your pallas kernel here")


# speedup vs baseline: 56.0759x; 56.0759x over previous
import jax
import jax.numpy as jnp
from jax.experimental import pallas as pl

D = 1024
M = 64
B = 16
K = 2
K_DRAFT = 4
LAMBDA = 0.1
EPS = 1e-12

_F32 = jnp.float32
_HI = jax.lax.Precision.HIGHEST


def _prep_kernel(vt3_ref, ut_ref, us_ref, g_ref, ueff_ref):
    vt3 = vt3_ref[...]
    nsq = jnp.sum(jnp.sum(vt3 * vt3, axis=2, keepdims=True), axis=1,
                  keepdims=True)
    vt3n = vt3 * (1.0 / jnp.maximum(jnp.sqrt(nsq), EPS))
    g = jnp.zeros((M, M), _F32)
    for b in range(B):
        vb = vt3n[:, b, :]
        g = g + jax.lax.dot_general(vb, vb, (((1,), (1,)), ((), ())),
                                    preferred_element_type=_F32)
    g_ref[...] = g

    ut = ut_ref[...]
    nsq_u = jnp.sum(ut * ut, axis=1, keepdims=True)
    su = jnp.tanh(us_ref[...]) * LAMBDA
    ueff_ref[...] = ut * (su / jnp.maximum(jnp.sqrt(nsq_u), EPS))


def _main_kernel(x_ref, v2n_ref, ueff_ref, g_ref, y_ref, raw_ref, ov_ref):
    nb = x_ref.shape[0]

    @pl.when(pl.program_id(0) == 0)
    def _init():
        raw_ref[...] = jnp.zeros_like(raw_ref)
        ov_ref[...] = jnp.zeros_like(ov_ref)

    xb = x_ref[...]
    r = jnp.dot(xb, v2n_ref[...], preferred_element_type=_F32)

    group = (jax.lax.broadcasted_iota(jnp.int32, (M * B, M), 0) // B
             == jax.lax.broadcasted_iota(jnp.int32, (M * B, M), 1)
             ).astype(_F32)
    e_sq = jnp.dot(r * r, group, preferred_element_type=_F32, precision=_HI)

    lane = jax.lax.broadcasted_iota(jnp.int32, (nb, M), 1)
    neg_inf = _F32(-jnp.inf)

    work = e_sq
    draftmask = jnp.zeros((nb, M), jnp.bool_)
    for _ in range(K_DRAFT):
        mx = jnp.max(work, axis=1, keepdims=True)
        idx = jnp.min(jnp.where(work == mx, lane, M), axis=1, keepdims=True)
        sel = lane == idx
        draftmask = jnp.logical_or(draftmask, sel)
        work = jnp.where(sel, neg_inf, work)

    e_draft = jnp.where(draftmask, jnp.sqrt(e_sq), 0.0)

    g = g_ref[...]
    eye = (jax.lax.broadcasted_iota(jnp.int32, (M, M), 0)
           == jax.lax.broadcasted_iota(jnp.int32, (M, M), 1))
    diag_g = jnp.sum(jnp.where(eye, g, 0.0), axis=0, keepdims=True)

    t = jnp.dot(e_draft, g, preferred_element_type=_F32)
    eb = e_draft.astype(jnp.bfloat16).astype(_F32)
    gdb = diag_g.astype(jnp.bfloat16).astype(_F32)
    inh = (t - eb * gdb) * e_draft
    corr = jnp.where(draftmask, e_sq - inh, neg_inf)

    work = corr
    finalmask = jnp.zeros((nb, M), jnp.bool_)
    for _ in range(K):
        mx = jnp.max(work, axis=1, keepdims=True)
        idx = jnp.min(jnp.where(work == mx, lane, M), axis=1, keepdims=True)
        sel = lane == idx
        finalmask = jnp.logical_or(finalmask, sel)
        work = jnp.where(sel, neg_inf, work)

    fmask = finalmask.astype(_F32)
    e_fin = e_draft * fmask

    raw_part = jnp.sum(e_sq * fmask)
    t2 = jnp.dot(e_fin, g, preferred_element_type=_F32)
    efb = e_fin.astype(jnp.bfloat16).astype(_F32)
    ov_part = jnp.sum(t2 * e_fin) - jnp.sum(efb * gdb * e_fin)
    raw_ref[...] += raw_part.reshape(1, 1)
    ov_ref[...] += ov_part.reshape(1, 1)

    expand = (jax.lax.broadcasted_iota(jnp.int32, (M, M * B), 0)
              == jax.lax.broadcasted_iota(jnp.int32, (M, M * B), 1) // B
              ).astype(_F32)
    mask_mb = jnp.dot(fmask, expand, preferred_element_type=_F32)
    writes = jnp.dot(r * mask_mb, ueff_ref[...], preferred_element_type=_F32)
    y_ref[...] = xb + writes


@jax.jit
def kernel(x, V_raw, U_raw, u_scales):
    n = x.shape[0] * x.shape[1]
    x_flat = x.reshape(n, D).astype(_F32)
    vn = V_raw / jnp.maximum(jnp.linalg.norm(V_raw, axis=0, keepdims=True),
                             EPS)
    v2n = vn.reshape(D, M * B)
    vt3 = jnp.transpose(vn, (1, 2, 0))
    ut = jnp.transpose(U_raw, (0, 2, 1)).reshape(M * B, D)
    us = u_scales.reshape(M * B, 1)

    g, ueff = pl.pallas_call(
        _prep_kernel,
        out_shape=[
            jax.ShapeDtypeStruct((M, M), _F32),
            jax.ShapeDtypeStruct((M * B, D), _F32),
        ],
    )(vt3, ut, us)

    nb = 256
    grid = (n // nb,)
    y, raw, ov = pl.pallas_call(
        _main_kernel,
        grid=grid,
        in_specs=[
            pl.BlockSpec((nb, D), lambda i: (i, 0)),
            pl.BlockSpec((D, M * B), lambda i: (0, 0)),
            pl.BlockSpec((M * B, D), lambda i: (0, 0)),
            pl.BlockSpec((M, M), lambda i: (0, 0)),
        ],
        out_specs=[
            pl.BlockSpec((nb, D), lambda i: (i, 0)),
            pl.BlockSpec((1, 1), lambda i: (0, 0)),
            pl.BlockSpec((1, 1), lambda i: (0, 0)),
        ],
        out_shape=[
            jax.ShapeDtypeStruct((n, D), _F32),
            jax.ShapeDtypeStruct((1, 1), _F32),
            jax.ShapeDtypeStruct((1, 1), _F32),
        ],
    )(x_flat, v2n, ueff, g)

    raw_b = raw[0, 0] / n
    ov_b = ov[0, 0] / n
    total = raw_b - ov_b
    aux = {
        'total_energy': total,
        'raw_energy': raw_b,
        'overlap_penalty': ov_b,
        'aux_loss': -total,
    }
    return y.reshape(x.shape), aux


# speedup vs baseline: 73.7822x; 1.3158x over previous
import jax
import jax.numpy as jnp
from jax.experimental import pallas as pl

D = 1024
M = 64
B = 16
K = 2
K_DRAFT = 4
LAMBDA = 0.1
EPS = 1e-12

_F32 = jnp.float32


def _prep_kernel(vt3_ref, ut_ref, us_ref, g_ref, ueff_ref):
    vt3 = vt3_ref[...]
    nsq = jnp.sum(jnp.sum(vt3 * vt3, axis=2, keepdims=True), axis=1,
                  keepdims=True)
    vt3n = vt3 * (1.0 / jnp.maximum(jnp.sqrt(nsq), EPS))
    g = jnp.zeros((M, M), _F32)
    for b in range(B):
        vb = vt3n[:, b, :]
        g = g + jax.lax.dot_general(vb, vb, (((1,), (1,)), ((), ())),
                                    preferred_element_type=_F32)
    g_ref[...] = g

    ut = ut_ref[...]
    nsq_u = jnp.sum(ut * ut, axis=1, keepdims=True)
    su = jnp.tanh(us_ref[...]) * LAMBDA
    ueff_ref[...] = ut * (su / jnp.maximum(jnp.sqrt(nsq_u), EPS))


def _main_kernel(x_ref, vb2_ref, ueff_ref, g_ref, y_ref, raw_ref, ov_ref):
    nb = x_ref.shape[0]

    @pl.when(pl.program_id(0) == 0)
    def _init():
        raw_ref[...] = jnp.zeros_like(raw_ref)
        ov_ref[...] = jnp.zeros_like(ov_ref)

    xb = x_ref[...]
    r = jnp.dot(xb, vb2_ref[...], preferred_element_type=_F32)

    s = jnp.zeros((nb, 2 * M), _F32)
    for b2 in range(B // 2):
        c = r[:, b2 * 2 * M:(b2 + 1) * 2 * M]
        s = s + c * c
    e_sq = s[:, :M] + s[:, M:]

    lane = jax.lax.broadcasted_iota(jnp.int32, (nb, M), 1)
    neg_inf = _F32(-jnp.inf)

    work = e_sq
    draftmask = jnp.zeros((nb, M), jnp.bool_)
    for _ in range(K_DRAFT):
        idx = jnp.argmax(work, axis=1, keepdims=True)
        sel = lane == idx
        draftmask = jnp.logical_or(draftmask, sel)
        work = jnp.where(sel, neg_inf, work)

    e_draft = jnp.where(draftmask, jnp.sqrt(e_sq), 0.0)

    g = g_ref[...]
    eye = (jax.lax.broadcasted_iota(jnp.int32, (M, M), 0)
           == jax.lax.broadcasted_iota(jnp.int32, (M, M), 1))
    diag_g = jnp.sum(jnp.where(eye, g, 0.0), axis=0, keepdims=True)

    t = jnp.dot(e_draft, g, preferred_element_type=_F32)
    eb = e_draft.astype(jnp.bfloat16).astype(_F32)
    gdb = diag_g.astype(jnp.bfloat16).astype(_F32)
    inh = (t - eb * gdb) * e_draft
    corr = jnp.where(draftmask, e_sq - inh, neg_inf)

    work = corr
    finalmask = jnp.zeros((nb, M), jnp.bool_)
    for _ in range(K):
        idx = jnp.argmax(work, axis=1, keepdims=True)
        sel = lane == idx
        finalmask = jnp.logical_or(finalmask, sel)
        work = jnp.where(sel, neg_inf, work)

    fmask = finalmask.astype(_F32)
    e_fin = e_draft * fmask

    raw_part = jnp.sum(e_sq * fmask)
    t2 = jnp.dot(e_fin, g, preferred_element_type=_F32)
    efb = e_fin.astype(jnp.bfloat16).astype(_F32)
    ov_part = jnp.sum(t2 * e_fin) - jnp.sum(efb * gdb * e_fin)
    raw_ref[...] += raw_part.reshape(1, 1)
    ov_ref[...] += ov_part.reshape(1, 1)

    expand = (jax.lax.broadcasted_iota(jnp.int32, (M, B * M), 0)
              == jax.lax.broadcasted_iota(jnp.int32, (M, B * M), 1) % M
              ).astype(_F32)
    mask_bm = jnp.dot(fmask, expand, preferred_element_type=_F32)
    writes = jnp.dot(r * mask_bm, ueff_ref[...], preferred_element_type=_F32)
    y_ref[...] = xb + writes


@jax.jit
def kernel(x, V_raw, U_raw, u_scales):
    n = x.shape[0] * x.shape[1]
    x_flat = x.reshape(n, D).astype(_F32)
    vn = V_raw / jnp.maximum(jnp.linalg.norm(V_raw, axis=0, keepdims=True),
                             EPS)
    vb2 = jnp.transpose(vn, (0, 2, 1)).reshape(D, B * M)
    vt3 = jnp.transpose(vn, (1, 2, 0))
    ut = jnp.transpose(U_raw, (2, 0, 1)).reshape(B * M, D)
    us = jnp.transpose(u_scales).reshape(B * M, 1)

    g, ueff = pl.pallas_call(
        _prep_kernel,
        out_shape=[
            jax.ShapeDtypeStruct((M, M), _F32),
            jax.ShapeDtypeStruct((B * M, D), _F32),
        ],
    )(vt3, ut, us)

    nb = 256
    grid = (n // nb,)
    y, raw, ov = pl.pallas_call(
        _main_kernel,
        grid=grid,
        in_specs=[
            pl.BlockSpec((nb, D), lambda i: (i, 0)),
            pl.BlockSpec((D, B * M), lambda i: (0, 0)),
            pl.BlockSpec((B * M, D), lambda i: (0, 0)),
            pl.BlockSpec((M, M), lambda i: (0, 0)),
        ],
        out_specs=[
            pl.BlockSpec((nb, D), lambda i: (i, 0)),
            pl.BlockSpec((1, 1), lambda i: (0, 0)),
            pl.BlockSpec((1, 1), lambda i: (0, 0)),
        ],
        out_shape=[
            jax.ShapeDtypeStruct((n, D), _F32),
            jax.ShapeDtypeStruct((1, 1), _F32),
            jax.ShapeDtypeStruct((1, 1), _F32),
        ],
    )(x_flat, vb2, ueff, g)

    raw_b = raw[0, 0] / n
    ov_b = ov[0, 0] / n
    total = raw_b - ov_b
    aux = {
        'total_energy': total,
        'raw_energy': raw_b,
        'overlap_penalty': ov_b,
        'aux_loss': -total,
    }
    return y.reshape(x.shape), aux


# speedup vs baseline: 89.2935x; 1.2102x over previous
import jax
import jax.numpy as jnp
from jax.experimental import pallas as pl
from jax.experimental.pallas import tpu as pltpu

D = 1024
M = 64
B = 16
K = 2
K_DRAFT = 4
LAMBDA = 0.1
EPS = 1e-12

_F32 = jnp.float32


def _fused_kernel(x_ref, vb2_ref, vt3_ref, ut_ref, us_ref,
                  y_ref, raw_ref, ov_ref, g_ref, ueff_ref):
    nb = x_ref.shape[0]

    @pl.when(pl.program_id(0) == 0)
    def _prep():
        vt3 = vt3_ref[...]
        nsq = jnp.sum(jnp.sum(vt3 * vt3, axis=2, keepdims=True), axis=1,
                      keepdims=True)
        vt3n = vt3 * (1.0 / jnp.maximum(jnp.sqrt(nsq), EPS))
        g = jnp.zeros((M, M), _F32)
        for b in range(B):
            vb = vt3n[:, b, :]
            g = g + jax.lax.dot_general(vb, vb, (((1,), (1,)), ((), ())),
                                        preferred_element_type=_F32)
        g_ref[...] = g

        ut = ut_ref[...]
        nsq_u = jnp.sum(ut * ut, axis=1, keepdims=True)
        su = jnp.tanh(us_ref[...]) * LAMBDA
        ueff_ref[...] = ut * (su / jnp.maximum(jnp.sqrt(nsq_u), EPS))

        raw_ref[...] = jnp.zeros_like(raw_ref)
        ov_ref[...] = jnp.zeros_like(ov_ref)

    xb = x_ref[...]
    r = jnp.dot(xb, vb2_ref[...], preferred_element_type=_F32)

    s = jnp.zeros((nb, 2 * M), _F32)
    for b2 in range(B // 2):
        c = r[:, b2 * 2 * M:(b2 + 1) * 2 * M]
        s = s + c * c
    e_sq = s[:, :M] + s[:, M:]

    lane = jax.lax.broadcasted_iota(jnp.int32, (nb, M), 1)
    neg_inf = _F32(-jnp.inf)

    work = e_sq
    draftmask = jnp.zeros((nb, M), jnp.bool_)
    for _ in range(K_DRAFT):
        idx = jnp.argmax(work, axis=1, keepdims=True)
        sel = lane == idx
        draftmask = jnp.logical_or(draftmask, sel)
        work = jnp.where(sel, neg_inf, work)

    e_draft = jnp.where(draftmask, jnp.sqrt(e_sq), 0.0)

    g = g_ref[...]
    eye = (jax.lax.broadcasted_iota(jnp.int32, (M, M), 0)
           == jax.lax.broadcasted_iota(jnp.int32, (M, M), 1))
    diag_g = jnp.sum(jnp.where(eye, g, 0.0), axis=0, keepdims=True)

    t = jnp.dot(e_draft, g, preferred_element_type=_F32)
    eb = e_draft.astype(jnp.bfloat16).astype(_F32)
    gdb = diag_g.astype(jnp.bfloat16).astype(_F32)
    inh = (t - eb * gdb) * e_draft
    corr = jnp.where(draftmask, e_sq - inh, neg_inf)

    work = corr
    finalmask = jnp.zeros((nb, M), jnp.bool_)
    for _ in range(K):
        idx = jnp.argmax(work, axis=1, keepdims=True)
        sel = lane == idx
        finalmask = jnp.logical_or(finalmask, sel)
        work = jnp.where(sel, neg_inf, work)

    fmask = finalmask.astype(_F32)
    e_fin = e_draft * fmask

    raw_part = jnp.sum(e_sq * fmask)
    t2 = jnp.dot(e_fin, g, preferred_element_type=_F32)
    efb = e_fin.astype(jnp.bfloat16).astype(_F32)
    ov_part = jnp.sum(t2 * e_fin) - jnp.sum(efb * gdb * e_fin)
    raw_ref[...] += raw_part.reshape(1, 1)
    ov_ref[...] += ov_part.reshape(1, 1)

    expand = (jax.lax.broadcasted_iota(jnp.int32, (M, B * M), 0)
              == jax.lax.broadcasted_iota(jnp.int32, (M, B * M), 1) % M
              ).astype(_F32)
    mask_bm = jnp.dot(fmask, expand, preferred_element_type=_F32)
    writes = jnp.dot(r * mask_bm, ueff_ref[...], preferred_element_type=_F32)
    y_ref[...] = xb + writes


@jax.jit
def kernel(x, V_raw, U_raw, u_scales):
    n = x.shape[0] * x.shape[1]
    x_flat = x.reshape(n, D).astype(_F32)
    vn = V_raw / jnp.maximum(jnp.linalg.norm(V_raw, axis=0, keepdims=True),
                             EPS)
    vb2 = jnp.transpose(vn, (0, 2, 1)).reshape(D, B * M)
    vt3 = jnp.transpose(vn, (1, 2, 0))
    ut = jnp.transpose(U_raw, (2, 0, 1)).reshape(B * M, D)
    us = jnp.transpose(u_scales).reshape(B * M, 1)

    nb = 512
    grid = (n // nb,)
    y, raw, ov = pl.pallas_call(
        _fused_kernel,
        grid=grid,
        in_specs=[
            pl.BlockSpec((nb, D), lambda i: (i, 0)),
            pl.BlockSpec((D, B * M), lambda i: (0, 0)),
            pl.BlockSpec((M, B, D), lambda i: (0, 0, 0)),
            pl.BlockSpec((B * M, D), lambda i: (0, 0)),
            pl.BlockSpec((B * M, 1), lambda i: (0, 0)),
        ],
        out_specs=[
            pl.BlockSpec((nb, D), lambda i: (i, 0)),
            pl.BlockSpec((1, 1), lambda i: (0, 0)),
            pl.BlockSpec((1, 1), lambda i: (0, 0)),
        ],
        out_shape=[
            jax.ShapeDtypeStruct((n, D), _F32),
            jax.ShapeDtypeStruct((1, 1), _F32),
            jax.ShapeDtypeStruct((1, 1), _F32),
        ],
        scratch_shapes=[
            pltpu.VMEM((M, M), _F32),
            pltpu.VMEM((B * M, D), _F32),
        ],
    )(x_flat, vb2, vt3, ut, us)

    raw_b = raw[0, 0] / n
    ov_b = ov[0, 0] / n
    total = raw_b - ov_b
    aux = {
        'total_energy': total,
        'raw_energy': raw_b,
        'overlap_penalty': ov_b,
        'aux_loss': -total,
    }
    return y.reshape(x.shape), aux
